# Initial kernel scaffold; baseline (speedup 1.0000x reference)
import jax
import jax.numpy as jnp
from jax.experimental import pallas as pl


def kernel(words, contexts, w_table, c_table):
    raise NotImplementedError("write your pallas kernel here")


# speedup vs baseline: 3.1894x; 3.1894x over previous
import functools

import jax
import jax.numpy as jnp
from jax import lax
from jax.experimental import pallas as pl
from jax.experimental.pallas import tpu as pltpu
from jax.experimental.pallas import tpu_sc as plsc

_CH = 128


@functools.lru_cache(maxsize=None)
def _make_gather(V, D, BW, BC):
    info = plsc.get_sparse_core_info()
    NC, NS = info.num_cores, info.num_subcores
    NW = NC * NS

    w_ch = BW // (NW * _CH)
    c_ch = BC // (NW * _CH)
    assert BW == NW * _CH * w_ch and BC == NW * _CH * c_ch
    assert w_ch % 2 == 0 and c_ch % 2 == 0

    mesh = plsc.VectorSubcoreMesh(core_axis_name="c", subcore_axis_name="s")

    @functools.partial(
        pl.kernel,
        mesh=mesh,
        out_type=[
            jax.ShapeDtypeStruct((BW, D), jnp.float32),
            jax.ShapeDtypeStruct((BC, D), jnp.float32),
        ],
        scratch_types=[
            pltpu.VMEM((w_ch, _CH), jnp.int32),
            pltpu.VMEM((c_ch, _CH), jnp.int32),
            pltpu.VMEM((_CH, D), jnp.float32),
            pltpu.VMEM((_CH, D), jnp.float32),
            pltpu.SemaphoreType.DMA,
            pltpu.SemaphoreType.DMA,
            pltpu.SemaphoreType.DMA,
            pltpu.SemaphoreType.DMA,
        ],
    )
    def gather_kernel(w_tab, c_tab, widx_hbm, cidx_hbm, w_out, c_out,
                      widx_v, cidx_v, rows0, rows1, g0, g1, s0, s1):
        wid = lax.axis_index("s") * NC + lax.axis_index("c")

        pltpu.sync_copy(widx_hbm.at[pl.ds(wid * w_ch, w_ch)], widx_v)
        pltpu.sync_copy(cidx_hbm.at[pl.ds(wid * c_ch, c_ch)], cidx_v)

        def phase(tab, idx_v, out, n_ch):
            base = wid * n_ch * _CH
            n_pairs = n_ch // 2

            pltpu.async_copy(tab.at[idx_v.at[0]], rows0, g0)

            def body(p, carry):
                j0 = 2 * p
                pltpu.make_async_copy(tab.at[idx_v.at[0]], rows0, g0).wait()

                @pl.when(p >= 1)
                def _():
                    pltpu.make_async_copy(rows1, out.at[pl.ds(base, _CH)], s1).wait()

                pltpu.async_copy(tab.at[idx_v.at[j0 + 1]], rows1, g1)
                pltpu.async_copy(rows0, out.at[pl.ds(base + j0 * _CH, _CH)], s0)

                pltpu.make_async_copy(tab.at[idx_v.at[0]], rows1, g1).wait()
                pltpu.make_async_copy(rows0, out.at[pl.ds(base, _CH)], s0).wait()

                @pl.when(p + 1 < n_pairs)
                def _():
                    pltpu.async_copy(tab.at[idx_v.at[j0 + 2]], rows0, g0)

                pltpu.async_copy(rows1, out.at[pl.ds(base + (j0 + 1) * _CH, _CH)], s1)
                return carry

            lax.fori_loop(0, n_pairs, body, 0)
            pltpu.make_async_copy(rows1, out.at[pl.ds(base, _CH)], s1).wait()

        phase(w_tab, widx_v, w_out, w_ch)
        phase(c_tab, cidx_v, c_out, c_ch)

    return gather_kernel


def kernel(words, contexts, w_table, c_table):
    B, = words.shape
    _, L = contexts.shape
    V, D = w_table.shape

    widx = words.astype(jnp.int32).reshape(B // _CH, _CH)
    cidx = contexts.astype(jnp.int32).reshape(B * L // _CH, _CH)

    w_out, c_out = _make_gather(V, D, B, B * L)(w_table, c_table, widx, cidx)
    return w_out, c_out.reshape(B, L, D)


# speedup vs baseline: 3.4338x; 1.0766x over previous
import functools

import jax
import jax.numpy as jnp
from jax import lax
from jax.experimental import pallas as pl
from jax.experimental.pallas import tpu as pltpu
from jax.experimental.pallas import tpu_sc as plsc

_CH = 128


@functools.lru_cache(maxsize=None)
def _make_gather(V, D, BW, BC):
    info = plsc.get_sparse_core_info()
    NC, NS = info.num_cores, info.num_subcores
    NW = NC * NS

    w_ch = BW // (NW * _CH)
    c_ch = BC // (NW * _CH)
    assert BW == NW * _CH * w_ch and BC == NW * _CH * c_ch
    assert w_ch % 2 == 0 and c_ch % 2 == 0

    mesh = plsc.VectorSubcoreMesh(core_axis_name="c", subcore_axis_name="s")

    @functools.partial(
        pl.kernel,
        mesh=mesh,
        out_type=[
            jax.ShapeDtypeStruct((BW, D), jnp.float32),
            jax.ShapeDtypeStruct((BC, D), jnp.float32),
        ],
        scratch_types=[
            pltpu.VMEM((w_ch, _CH), jnp.int32),
            pltpu.VMEM((c_ch, _CH), jnp.int32),
            pltpu.VMEM((_CH, D), jnp.float32),
            pltpu.VMEM((_CH, D), jnp.float32),
            pltpu.VMEM((_CH, D), jnp.float32),
            pltpu.VMEM((_CH, D), jnp.float32),
            pltpu.SemaphoreType.DMA,
            pltpu.SemaphoreType.DMA,
            pltpu.SemaphoreType.DMA,
            pltpu.SemaphoreType.DMA,
            pltpu.SemaphoreType.DMA,
            pltpu.SemaphoreType.DMA,
            pltpu.SemaphoreType.DMA,
            pltpu.SemaphoreType.DMA,
        ],
    )
    def gather_kernel(w_tab, c_tab, widx_hbm, cidx_hbm, w_out, c_out,
                      widx_v, cidx_v, r0, r1, r2, r3,
                      g0, g1, g2, g3, s0, s1, s2, s3):
        wid = lax.axis_index("s") * NC + lax.axis_index("c")
        bufs = (r0, r1, r2, r3)
        gsems = (g0, g1, g2, g3)
        ssems = (s0, s1, s2, s3)

        pltpu.sync_copy(widx_hbm.at[pl.ds(wid * w_ch, w_ch)], widx_v)
        pltpu.sync_copy(cidx_hbm.at[pl.ds(wid * c_ch, c_ch)], cidx_v)

        def phase(tab, idx_v, out, n_ch):
            base = wid * n_ch * _CH
            P = n_ch // 4

            for b in range(3):
                pltpu.async_copy(tab.at[idx_v.at[b]], bufs[b], gsems[b])

            def body(p, carry):
                j0 = 4 * p
                for b in range(4):
                    j = j0 + b
                    pltpu.make_async_copy(tab.at[idx_v.at[0]], bufs[b], gsems[b]).wait()
                    pltpu.async_copy(bufs[b], out.at[pl.ds(base + j * _CH, _CH)], ssems[b])
                    if b == 0:
                        @pl.when(p >= 1)
                        def _():
                            pltpu.make_async_copy(bufs[3], out.at[pl.ds(base, _CH)], ssems[3]).wait()
                        pltpu.async_copy(tab.at[idx_v.at[j + 3]], bufs[3], gsems[3])
                    else:
                        @pl.when(p + 1 < P)
                        def _():
                            pltpu.make_async_copy(bufs[b - 1], out.at[pl.ds(base, _CH)], ssems[b - 1]).wait()
                            pltpu.async_copy(tab.at[idx_v.at[j + 3]], bufs[b - 1], gsems[b - 1])
                return carry

            lax.fori_loop(0, P, body, 0)
            for b in range(4):
                pltpu.make_async_copy(bufs[b], out.at[pl.ds(base, _CH)], ssems[b]).wait()

        phase(w_tab, widx_v, w_out, w_ch)
        phase(c_tab, cidx_v, c_out, c_ch)

    return gather_kernel


def kernel(words, contexts, w_table, c_table):
    B, = words.shape
    _, L = contexts.shape
    V, D = w_table.shape

    widx = words.astype(jnp.int32).reshape(B // _CH, _CH)
    cidx = contexts.astype(jnp.int32).reshape(B * L // _CH, _CH)

    w_out, c_out = _make_gather(V, D, B, B * L)(w_table, c_table, widx, cidx)
    return w_out, c_out.reshape(B, L, D)


# speedup vs baseline: 5.7083x; 1.6624x over previous
import functools

import jax
import jax.numpy as jnp
from jax import lax
from jax.experimental import pallas as pl
from jax.experimental.pallas import tpu as pltpu
from jax.experimental.pallas import tpu_sc as plsc


@functools.lru_cache(maxsize=None)
def _make_gather(V, D, B, L):
    info = plsc.get_sparse_core_info()
    NC, NS = info.num_cores, info.num_subcores
    NW = NC * NS

    WPW = B // NW
    CW = 4
    G = CW * L
    c_ch = WPW // CW
    WC = 64
    w_ch = WPW // WC
    assert w_ch % 4 == 0 and c_ch % 4 == 0 and G <= 128 and WC <= G

    mesh = plsc.VectorSubcoreMesh(core_axis_name="c", subcore_axis_name="s")

    @functools.partial(
        pl.kernel,
        mesh=mesh,
        out_type=[
            jax.ShapeDtypeStruct((B, D), jnp.float32),
            jax.ShapeDtypeStruct((B, L, D), jnp.float32),
        ],
        scratch_types=[
            pltpu.VMEM((WPW,), jnp.int32),
            pltpu.VMEM((WPW * L,), jnp.int32),
            pltpu.VMEM((G, D), jnp.float32),
            pltpu.VMEM((G, D), jnp.float32),
            pltpu.VMEM((G, D), jnp.float32),
            pltpu.VMEM((G, D), jnp.float32),
            pltpu.SemaphoreType.DMA,
            pltpu.SemaphoreType.DMA,
            pltpu.SemaphoreType.DMA,
            pltpu.SemaphoreType.DMA,
            pltpu.SemaphoreType.DMA,
            pltpu.SemaphoreType.DMA,
            pltpu.SemaphoreType.DMA,
            pltpu.SemaphoreType.DMA,
        ],
    )
    def gather_kernel(w_tab, c_tab, widx_hbm, cidx_hbm, w_out, c_out,
                      widx_v, cidx_v, r0, r1, r2, r3,
                      g0, g1, g2, g3, s0, s1, s2, s3):
        wid = lax.axis_index("s") * NC + lax.axis_index("c")
        bufs = (r0, r1, r2, r3)
        gsems = (g0, g1, g2, g3)
        ssems = (s0, s1, s2, s3)

        pltpu.sync_copy(widx_hbm.at[pl.ds(wid * WPW, WPW)], widx_v)
        pltpu.sync_copy(cidx_hbm.at[pl.ds(wid * WPW * L, WPW * L)], cidx_v)

        def ring(n_ch, issue_gather, wait_gather, issue_scatter, wait_scatter):
            P = n_ch // 4
            for b in range(3):
                issue_gather(b, b)

            def body(p, carry):
                j0 = 4 * p
                for b in range(4):
                    j = j0 + b
                    wait_gather(b)
                    issue_scatter(j, b)
                    if b == 0:
                        @pl.when(p >= 1)
                        def _():
                            wait_scatter(3)
                        issue_gather(j + 3, 3)
                    else:
                        @pl.when(p + 1 < P)
                        def _():
                            wait_scatter(b - 1)
                            issue_gather(j + 3, b - 1)
                return carry

            lax.fori_loop(0, P, body, 0)
            for b in range(4):
                wait_scatter(b)

        wbase = wid * WPW

        def w_ig(j, b):
            pltpu.async_copy(w_tab.at[widx_v.at[pl.ds(j * WC, WC)]],
                             bufs[b].at[pl.ds(0, WC)], gsems[b])

        def w_wg(b):
            pltpu.make_async_copy(w_tab.at[widx_v.at[pl.ds(0, WC)]],
                                  bufs[b].at[pl.ds(0, WC)], gsems[b]).wait()

        def w_is(j, b):
            pltpu.async_copy(bufs[b].at[pl.ds(0, WC)],
                             w_out.at[pl.ds(wbase + j * WC, WC)], ssems[b])

        def w_ws(b):
            pltpu.make_async_copy(bufs[b].at[pl.ds(0, WC)],
                                  w_out.at[pl.ds(wbase, WC)], ssems[b]).wait()

        ring(w_ch, w_ig, w_wg, w_is, w_ws)

        def c_ig(j, b):
            pltpu.async_copy(c_tab.at[cidx_v.at[pl.ds(j * G, G)]], bufs[b], gsems[b])

        def c_wg(b):
            pltpu.make_async_copy(c_tab.at[cidx_v.at[pl.ds(0, G)]], bufs[b], gsems[b]).wait()

        def c_is(j, b):
            for w in range(CW):
                pltpu.async_copy(bufs[b].at[pl.ds(w * L, L)],
                                 c_out.at[wbase + j * CW + w], ssems[b])

        def c_ws(b):
            for w in range(CW):
                pltpu.make_async_copy(bufs[b].at[pl.ds(0, L)], c_out.at[0], ssems[b]).wait()

        ring(c_ch, c_ig, c_wg, c_is, c_ws)

    return gather_kernel


def kernel(words, contexts, w_table, c_table):
    B, = words.shape
    _, L = contexts.shape
    V, D = w_table.shape

    widx = words.astype(jnp.int32)
    cidx = contexts.astype(jnp.int32).reshape(B * L)

    w_out, c_out = _make_gather(V, D, B, L)(w_table, c_table, widx, cidx)
    return w_out, c_out


# speedup vs baseline: 10.8986x; 1.9093x over previous
import functools

import jax
import jax.numpy as jnp
from jax import lax
from jax.experimental import pallas as pl
from jax.experimental.pallas import tpu as pltpu
from jax.experimental.pallas import tpu_sc as plsc

_CH = 128


@functools.lru_cache(maxsize=None)
def _make_gather(V, D, BW, BC):
    info = plsc.get_sparse_core_info()
    NC, NS = info.num_cores, info.num_subcores
    NW = NC * NS

    WPW = BW // NW
    CPW = BC // NW
    w_ch = WPW // _CH
    c_ch = CPW // _CH
    assert WPW * NW == BW and CPW * NW == BC
    assert w_ch % 4 == 0 and c_ch % 4 == 0

    mesh = plsc.VectorSubcoreMesh(core_axis_name="c", subcore_axis_name="s")

    @functools.partial(
        pl.kernel,
        mesh=mesh,
        out_type=[
            jax.ShapeDtypeStruct((BW, D), jnp.float32),
            jax.ShapeDtypeStruct((BC, D), jnp.float32),
        ],
        scratch_types=[
            pltpu.VMEM((WPW,), jnp.int32),
            pltpu.VMEM((CPW,), jnp.int32),
            pltpu.VMEM((_CH, D), jnp.float32),
            pltpu.VMEM((_CH, D), jnp.float32),
            pltpu.VMEM((_CH, D), jnp.float32),
            pltpu.VMEM((_CH, D), jnp.float32),
            pltpu.SemaphoreType.DMA,
            pltpu.SemaphoreType.DMA,
            pltpu.SemaphoreType.DMA,
            pltpu.SemaphoreType.DMA,
            pltpu.SemaphoreType.DMA,
            pltpu.SemaphoreType.DMA,
            pltpu.SemaphoreType.DMA,
            pltpu.SemaphoreType.DMA,
        ],
    )
    def gather_kernel(w_tab, c_tab, widx_hbm, cidx_hbm, w_out, c_out,
                      widx_v, cidx_v, r0, r1, r2, r3,
                      g0, g1, g2, g3, s0, s1, s2, s3):
        wid = lax.axis_index("s") * NC + lax.axis_index("c")
        bufs = (r0, r1, r2, r3)
        gsems = (g0, g1, g2, g3)
        ssems = (s0, s1, s2, s3)

        pltpu.sync_copy(widx_hbm.at[pl.ds(wid * WPW, WPW)], widx_v)
        pltpu.sync_copy(cidx_hbm.at[pl.ds(wid * CPW, CPW)], cidx_v)

        def ring(n_ch, issue_gather, wait_gather, issue_scatter, wait_scatter):
            P = n_ch // 4
            for b in range(3):
                issue_gather(b, b)

            def body(p, carry):
                j0 = 4 * p
                for b in range(4):
                    j = j0 + b
                    wait_gather(b)
                    issue_scatter(j, b)
                    if b == 0:
                        @pl.when(p >= 1)
                        def _():
                            wait_scatter(3)
                        issue_gather(j + 3, 3)
                    else:
                        @pl.when(p + 1 < P)
                        def _():
                            wait_scatter(b - 1)
                            issue_gather(j + 3, b - 1)
                return carry

            lax.fori_loop(0, P, body, 0)
            for b in range(4):
                wait_scatter(b)

        def phase(tab, idx_v, out, base, n_ch):
            def ig(j, b):
                pltpu.async_copy(tab.at[idx_v.at[pl.ds(j * _CH, _CH)]], bufs[b], gsems[b])

            def wg(b):
                pltpu.make_async_copy(tab.at[idx_v.at[pl.ds(0, _CH)]], bufs[b], gsems[b]).wait()

            def isc(j, b):
                pltpu.async_copy(bufs[b], out.at[pl.ds(base + j * _CH, _CH)], ssems[b])

            def wsc(b):
                pltpu.make_async_copy(bufs[b], out.at[pl.ds(base, _CH)], ssems[b]).wait()

            ring(n_ch, ig, wg, isc, wsc)

        phase(w_tab, widx_v, w_out, wid * WPW, w_ch)
        phase(c_tab, cidx_v, c_out, wid * CPW, c_ch)

    return gather_kernel


def kernel(words, contexts, w_table, c_table):
    B, = words.shape
    _, L = contexts.shape
    V, D = w_table.shape

    widx = words.astype(jnp.int32)
    cidx = contexts.astype(jnp.int32).T.reshape(B * L)

    w_out, c2d = _make_gather(V, D, B, B * L)(w_table, c_table, widx, cidx)
    return w_out, c2d.reshape(L, B, D).transpose(1, 0, 2)


# speedup vs baseline: 10.9139x; 1.0014x over previous
import functools

import jax
import jax.numpy as jnp
from jax import lax
from jax.experimental import pallas as pl
from jax.experimental.pallas import tpu as pltpu
from jax.experimental.pallas import tpu_sc as plsc

_CH = 128


@functools.lru_cache(maxsize=None)
def _make_gather(V, D, BW, BC):
    info = plsc.get_sparse_core_info()
    NC, NS = info.num_cores, info.num_subcores
    NW = NC * NS

    WPW = BW // NW
    CPW = BC // NW
    w_ch = WPW // _CH
    c_ch = CPW // _CH
    K = 5
    assert WPW * NW == BW and CPW * NW == BC
    assert w_ch <= K - 1 and c_ch % K == 0

    mesh = plsc.VectorSubcoreMesh(core_axis_name="c", subcore_axis_name="s")

    @functools.partial(
        pl.kernel,
        mesh=mesh,
        out_type=[
            jax.ShapeDtypeStruct((BW, D), jnp.float32),
            jax.ShapeDtypeStruct((BC, D), jnp.float32),
        ],
        scratch_types=[
            pltpu.VMEM((WPW,), jnp.int32),
            pltpu.VMEM((CPW,), jnp.int32),
            pltpu.VMEM((_CH, D), jnp.float32),
            pltpu.VMEM((_CH, D), jnp.float32),
            pltpu.VMEM((_CH, D), jnp.float32),
            pltpu.VMEM((_CH, D), jnp.float32),
            pltpu.VMEM((_CH, D), jnp.float32),
            pltpu.SemaphoreType.DMA,
            pltpu.SemaphoreType.DMA,
            pltpu.SemaphoreType.DMA,
            pltpu.SemaphoreType.DMA,
            pltpu.SemaphoreType.DMA,
            pltpu.SemaphoreType.DMA,
            pltpu.SemaphoreType.DMA,
            pltpu.SemaphoreType.DMA,
            pltpu.SemaphoreType.DMA,
            pltpu.SemaphoreType.DMA,
        ],
    )
    def gather_kernel(w_tab, c_tab, widx_hbm, cidx_hbm, w_out, c_out,
                      widx_v, cidx_v, r0, r1, r2, r3, r4,
                      g0, g1, g2, g3, g4, s0, s1, s2, s3, s4):
        wid = lax.axis_index("s") * NC + lax.axis_index("c")
        bufs = (r0, r1, r2, r3, r4)
        gsems = (g0, g1, g2, g3, g4)
        ssems = (s0, s1, s2, s3, s4)

        pltpu.sync_copy(widx_hbm.at[pl.ds(wid * WPW, WPW)], widx_v)
        pltpu.sync_copy(cidx_hbm.at[pl.ds(wid * CPW, CPW)], cidx_v)

        def ring(n_ch, issue_gather, wait_gather, issue_scatter, wait_scatter):
            P = n_ch // K
            for b in range(K - 1):
                issue_gather(b, b)

            def body(p, carry):
                j0 = K * p
                for b in range(K):
                    j = j0 + b
                    wait_gather(b)
                    issue_scatter(j, b)
                    if b == 0:
                        @pl.when(p >= 1)
                        def _():
                            wait_scatter(K - 1)
                        issue_gather(j + K - 1, K - 1)
                    else:
                        @pl.when(p + 1 < P)
                        def _():
                            wait_scatter(b - 1)
                            issue_gather(j + K - 1, b - 1)
                return carry

            lax.fori_loop(0, P, body, 0)
            for b in range(K):
                wait_scatter(b)

        def smallphase(n_ch, issue_gather, wait_gather, issue_scatter, wait_scatter):
            for b in range(n_ch):
                issue_gather(b, b)
            for b in range(n_ch):
                wait_gather(b)
                issue_scatter(b, b)
            for b in range(n_ch):
                wait_scatter(b)

        def phase(tab, idx_v, out, base, n_ch, runner):
            def ig(j, b):
                pltpu.async_copy(tab.at[idx_v.at[pl.ds(j * _CH, _CH)]], bufs[b], gsems[b])

            def wg(b):
                pltpu.make_async_copy(tab.at[idx_v.at[pl.ds(0, _CH)]], bufs[b], gsems[b]).wait()

            def isc(j, b):
                pltpu.async_copy(bufs[b], out.at[pl.ds(base + j * _CH, _CH)], ssems[b])

            def wsc(b):
                pltpu.make_async_copy(bufs[b], out.at[pl.ds(base, _CH)], ssems[b]).wait()

            runner(n_ch, ig, wg, isc, wsc)

        phase(w_tab, widx_v, w_out, wid * WPW, w_ch, smallphase)
        phase(c_tab, cidx_v, c_out, wid * CPW, c_ch, ring)

    return gather_kernel


def kernel(words, contexts, w_table, c_table):
    B, = words.shape
    _, L = contexts.shape
    V, D = w_table.shape

    widx = words.astype(jnp.int32)
    cidx = contexts.astype(jnp.int32).T.reshape(B * L)

    w_out, c2d = _make_gather(V, D, B, B * L)(w_table, c_table, widx, cidx)
    return w_out, c2d.reshape(L, B, D).transpose(1, 0, 2)
